# baseline (speedup 1.0000x reference)
import functools

import jax
import jax.numpy as jnp
from jax import lax
from jax.experimental import pallas as pl
from jax.experimental.pallas import tpu as pltpu
from jax.experimental.pallas import tpu_sc as plsc

_L = 16

_GDN = jax.lax.GatherDimensionNumbers(
    offset_dims=(), collapsed_slice_dims=(0,), start_index_map=(0,))


def _allsum(v):
    for shift in (8, 4, 2, 1):
        idx = (lax.iota(jnp.int32, _L) + shift) & (_L - 1)
        rot = lax.gather(v, idx[:, None], _GDN, (1,),
                         mode=lax.GatherScatterMode.PROMISE_IN_BOUNDS)
        v = v + rot
    return v


def _rsqrt(v):
    i = lax.bitcast_convert_type(v, jnp.int32)
    i = jnp.int32(0x5F3759DF) - (i >> 1)
    y = lax.bitcast_convert_type(i, jnp.float32)
    for _ in range(3):
        y = y * (1.5 - 0.5 * v * y * y)
    return y


def kernel(x, mask):
    b, c, h, w = x.shape
    hw = h * w
    nrows = b * c
    x2 = x.reshape(nrows, hw)
    m2 = mask.reshape(b, hw)

    nw = 32
    rpw = nrows // nw
    rows_per_batch = c
    nchunk = hw // _L

    mesh = plsc.VectorSubcoreMesh(
        core_axis_name="c", subcore_axis_name="s", num_cores=2, num_subcores=16
    )

    @functools.partial(
        pl.kernel,
        out_type=jax.ShapeDtypeStruct((nrows, hw), jnp.float32),
        mesh=mesh,
        scratch_types=[
            pltpu.VMEM((hw,), jnp.float32),
            pltpu.VMEM((hw,), jnp.float32),
            pltpu.SemaphoreType.DMA,
        ],
    )
    def k(x_hbm, m_hbm, o_hbm, m_v, x_v, sem):
        wid = lax.axis_index("s") * 2 + lax.axis_index("c")
        base = wid * rpw
        bidx = base // rows_per_batch
        pltpu.sync_copy(m_hbm.at[bidx], m_v)

        def count_body(t, acc):
            m = m_v[pl.ds(t * _L, _L)]
            return acc + jnp.where(m >= 0.5, 1.0, 0.0)

        nf_vec = lax.fori_loop(
            0, nchunk, count_body, jnp.zeros((_L,), jnp.float32))
        n_fg = _allsum(nf_vec)
        n_bg = float(hw) - n_fg

        def row_body(r, carry):
            row = base + r
            pltpu.sync_copy(x_hbm.at[row], x_v)

            def stats_body(t, ac):
                sa, qa, sf, qf = ac
                xc = x_v[pl.ds(t * _L, _L)]
                m = m_v[pl.ds(t * _L, _L)]
                fgm = m >= 0.5
                sq = xc * xc
                return (sa + xc, qa + sq,
                        sf + jnp.where(fgm, xc, 0.0),
                        qf + jnp.where(fgm, sq, 0.0))

            z = jnp.zeros((_L,), jnp.float32)
            sa, qa, sf, qf = lax.fori_loop(
                0, nchunk, stats_body, (z, z, z, z))
            s_all = _allsum(sa)
            q_all = _allsum(qa)
            s_fg = _allsum(sf)
            q_fg = _allsum(qf)

            mu_fg = s_fg / n_fg
            mu_bg = (s_all - s_fg) / n_bg
            var_fg = (q_fg - n_fg * mu_fg * mu_fg) / (n_fg - 1.0)
            var_bg = ((q_all - q_fg) - n_bg * mu_bg * mu_bg) / (n_bg - 1.0)
            sig_fg = var_fg * _rsqrt(var_fg)
            sig_bg = var_bg * _rsqrt(var_bg)
            scale = sig_fg / (sig_bg + 1e-8)
            shift = mu_fg - scale * mu_bg

            def write_body(t, _):
                xc = x_v[pl.ds(t * _L, _L)]
                m = m_v[pl.ds(t * _L, _L)]
                y = xc * scale + shift
                x_v[pl.ds(t * _L, _L)] = jnp.where(m >= 0.5, xc, y)
                return 0

            lax.fori_loop(0, nchunk, write_body, 0)
            pltpu.sync_copy(x_v, o_hbm.at[row])
            return carry

        lax.fori_loop(0, rpw, row_body, 0)

    out = k(x2, m2)
    return out.reshape(b, c, h, w)


# speedup vs baseline: 5.9895x; 5.9895x over previous
import functools

import jax
import jax.numpy as jnp
from jax.experimental import pallas as pl


def _body(x_ref, m_ref, o_ref, *, hw, r, s, w, rows_per_b):
    bidx = pl.program_id(0) // rows_per_b
    ch = 8
    k_steps = s // ch
    zero = jnp.zeros((r, ch, w), jnp.float32)
    zrow = jnp.zeros((ch, w), jnp.float32)

    def stats_body(k, carry):
        sa, qa, sf, qf, nf = carry
        off = pl.multiple_of(k * ch, ch)
        m = m_ref[bidx, pl.ds(off, ch), :]
        fgm = m >= 0.5
        xc = x_ref[:, pl.ds(off, ch), :]
        sq = xc * xc
        xm = jnp.where(fgm, xc, 0.0)
        qm = jnp.where(fgm, sq, 0.0)
        nf = nf + jnp.where(fgm, 1.0, 0.0)
        return sa + xc, qa + sq, sf + xm, qf + qm, nf

    sa, qa, sf, qf, nf = jax.lax.fori_loop(
        0, k_steps, stats_body, (zero, zero, zero, zero, zrow))

    n_fg = jnp.sum(nf)
    n_bg = hw - n_fg
    s_all = jnp.sum(sa, axis=(1, 2))
    s_fg = jnp.sum(sf, axis=(1, 2))
    q_all = jnp.sum(qa, axis=(1, 2))
    q_fg = jnp.sum(qf, axis=(1, 2))

    mu_fg = s_fg / n_fg
    mu_bg = (s_all - s_fg) / n_bg
    var_fg = (q_fg - n_fg * mu_fg * mu_fg) / (n_fg - 1.0)
    var_bg = ((q_all - q_fg) - n_bg * mu_bg * mu_bg) / (n_bg - 1.0)
    scale = jnp.sqrt(var_fg) / (jnp.sqrt(var_bg) + 1e-8)
    shift = (mu_fg - scale * mu_bg)[:, None, None]
    scale = scale[:, None, None]

    def write_body(k, _):
        off = pl.multiple_of(k * ch, ch)
        m = m_ref[bidx, pl.ds(off, ch), :]
        fgm = m >= 0.5
        xc = x_ref[:, pl.ds(off, ch), :]
        y = xc * scale + shift
        o_ref[:, pl.ds(off, ch), :] = jnp.where(fgm, xc, y)
        return 0

    jax.lax.fori_loop(0, k_steps, write_body, 0)


def kernel(x, mask):
    b, c, h, w = x.shape
    hw = h * w
    x3 = x.reshape(b * c, h, w)
    m3 = mask.reshape(b, h, w)

    r = 8 if c % 8 == 0 else 1
    grid = (b * c) // r
    rows_per_b = c // r

    out = pl.pallas_call(
        functools.partial(_body, hw=float(hw), r=r, s=h, w=w, rows_per_b=rows_per_b),
        grid=(grid,),
        in_specs=[
            pl.BlockSpec((r, h, w), lambda i: (i, 0, 0)),
            pl.BlockSpec((b, h, w), lambda i: (0, 0, 0)),
        ],
        out_specs=pl.BlockSpec((r, h, w), lambda i: (i, 0, 0)),
        out_shape=jax.ShapeDtypeStruct((b * c, h, w), x.dtype),
    )(x3, m3)
    return out.reshape(b, c, h, w)
